# Initial kernel scaffold; baseline (speedup 1.0000x reference)
import jax
import jax.numpy as jnp
from jax.experimental import pallas as pl


def kernel(feat, coordinate, velocity, edge_index, W_e1, b_e1, W_e2, b_e2, W_c1, b_c1, W_c2, b_c2, W_n1, b_n1, W_n2, b_n2, W_v1, b_v1, W_v2, b_v2):
    raise NotImplementedError("write your pallas kernel here")


# speedup vs baseline: 3.3712x; 3.3712x over previous
import functools

import jax
import jax.numpy as jnp
from jax import lax
from jax.experimental import pallas as pl
from jax.experimental.pallas import tpu as pltpu
from jax.experimental.pallas import tpu_sc as plsc

N = 10000
E = 320000
F = 128
TW = 144
NC, NS, L = 2, 16, 16
NW = NC * NS
IDXW = 128
IDXROWS = E // IDXW
NB = 1000
EB = 2000
NPT = N // NS

_mesh = plsc.VectorSubcoreMesh(core_axis_name="c", subcore_axis_name="s")
_sc_params = pltpu.CompilerParams(use_tc_tiling_on_sc=False)


def _tables_body(feat_ref, coord_ref, wa_ref, wb_ref, ta_ref, tb_ref):
    f = feat_ref[...]
    a = jnp.dot(f, wa_ref[...], preferred_element_type=jnp.float32)
    b = jnp.dot(f, wb_ref[...], preferred_element_type=jnp.float32)
    c = coord_ref[...]
    pad = jnp.zeros((f.shape[0], TW - F - 3), jnp.float32)
    ta_ref[...] = jnp.concatenate([a, c, pad], axis=1)
    tb_ref[...] = jnp.concatenate([b, -c, pad], axis=1)


def _build_tables(feat, coordinate, wa, wb):
    return pl.pallas_call(
        _tables_body,
        grid=(N // NB,),
        in_specs=[
            pl.BlockSpec((NB, F), lambda i: (i, 0)),
            pl.BlockSpec((NB, 3), lambda i: (i, 0)),
            pl.BlockSpec((F, F), lambda i: (0, 0)),
            pl.BlockSpec((F, F), lambda i: (0, 0)),
        ],
        out_specs=[
            pl.BlockSpec((NB, TW), lambda i: (i, 0)),
            pl.BlockSpec((NB, TW), lambda i: (i, 0)),
        ],
        out_shape=[
            jax.ShapeDtypeStruct((N, TW), jnp.float32),
            jax.ShapeDtypeStruct((N, TW), jnp.float32),
        ],
    )(feat, coordinate, wa, wb)


def _gather_body(ta_hbm, tb_hbm, src_hbm, dst_hbm, out_hbm,
                 idx_s, idx_d, bufa, bufb, bufo, sem_a, sem_b):
    wid = lax.axis_index("s") * NC + lax.axis_index("c")

    @pl.loop(wid, IDXROWS, step=NW)
    def _(i):
        pltpu.sync_copy(src_hbm.at[pl.ds(i, 1)], idx_s)
        pltpu.sync_copy(dst_hbm.at[pl.ds(i, 1)], idx_d)
        cpa = pltpu.async_copy(ta_hbm.at[idx_s.at[0]], bufa, sem_a)
        cpb = pltpu.async_copy(tb_hbm.at[idx_d.at[0]], bufb, sem_b)
        cpa.wait()
        cpb.wait()

        @pl.loop(0, IDXW, step=8)
        def _(r0):
            for dr in range(8):
                for cc in range(TW // L):
                    sl = (r0 + dr, pl.ds(cc * L, L))
                    bufo[sl] = bufa[sl] + bufb[sl]

        pltpu.sync_copy(bufo, out_hbm.at[pl.ds(i * IDXW, IDXW)])


def _gather(ta, tb, src, dst):
    kfn = pl.kernel(
        _gather_body,
        out_type=jax.ShapeDtypeStruct((E, TW), jnp.float32),
        mesh=_mesh,
        scratch_types=[
            pltpu.VMEM((1, IDXW), jnp.int32),
            pltpu.VMEM((1, IDXW), jnp.int32),
            pltpu.VMEM((IDXW, TW), jnp.float32),
            pltpu.VMEM((IDXW, TW), jnp.float32),
            pltpu.VMEM((IDXW, TW), jnp.float32),
            pltpu.SemaphoreType.DMA,
            pltpu.SemaphoreType.DMA,
        ],
        compiler_params=_sc_params,
    )
    return kfn(ta, tb, src, dst)


def _edge_body(s_ref, be1_ref, we2_ref, be2_ref, wc1_ref, bc1_ref,
               wc2_ref, bc2_ref, wd2_ref, msg_ref):
    s = s_ref[...]
    sv = s[:, :F]
    dx = s[:, F:F + 3]
    d2 = jnp.sum(dx * dx, axis=1, keepdims=True)
    pre1 = sv + d2 * wd2_ref[...] + be1_ref[...]
    h1 = pre1 * jax.nn.sigmoid(pre1)
    pre2 = jnp.dot(h1, we2_ref[...], preferred_element_type=jnp.float32) + be2_ref[...]
    he = pre2 * jax.nn.sigmoid(pre2)
    pre3 = jnp.dot(he, wc1_ref[...], preferred_element_type=jnp.float32) + bc1_ref[...]
    hc = pre3 * jax.nn.sigmoid(pre3)
    c = jnp.dot(hc, wc2_ref[...], preferred_element_type=jnp.float32) + bc2_ref[0, 0]
    xe = dx * c
    pad = jnp.zeros((s.shape[0], TW - F - 3), jnp.float32)
    msg_ref[...] = jnp.concatenate([he, xe, pad], axis=1)


def _edge_mlp(s, be1, we2, be2, wc1, bc1, wc2, bc2, wd2):
    full = lambda shape: pl.BlockSpec(shape, lambda i: tuple(0 for _ in shape))
    return pl.pallas_call(
        _edge_body,
        grid=(E // EB,),
        in_specs=[
            pl.BlockSpec((EB, TW), lambda i: (i, 0)),
            full((1, F)), full((F, F)), full((1, F)), full((F, F)),
            full((1, F)), full((F, 1)), full((1, 1)), full((1, F)),
        ],
        out_specs=pl.BlockSpec((EB, TW), lambda i: (i, 0)),
        out_shape=jax.ShapeDtypeStruct((E, TW), jnp.float32),
    )(s, be1, we2, be2, wc1, bc1, wc2, bc2, wd2)


def _scatter_body(msg_hbm, dst_hbm, out_hbm, idxb, mbuf, acc):
    cid = lax.axis_index("c")
    sid = lax.axis_index("s")
    wid = sid * NC + cid
    base = sid * NPT

    @pl.loop(0, IDXW)
    def _(r):
        for cc in range(TW // L):
            mbuf[r, pl.ds(cc * L, L)] = jnp.zeros((L,), jnp.float32)

    @pl.loop(0, NPT // IDXW)
    def _(j):
        pltpu.sync_copy(mbuf, acc.at[pl.ds(base + j * IDXW, IDXW)])

    rem = NPT % IDXW
    pltpu.sync_copy(mbuf.at[pl.ds(0, rem)],
                    acc.at[pl.ds(base + NPT - rem, rem)])
    plsc.subcore_barrier()

    @pl.loop(wid, IDXROWS, step=NW)
    def _(i):
        pltpu.sync_copy(dst_hbm.at[pl.ds(i, 1)], idxb)
        pltpu.sync_copy(msg_hbm.at[pl.ds(i * IDXW, IDXW)], mbuf)
        pltpu.sync_copy(mbuf, acc.at[idxb.at[0]], add=True)

    plsc.subcore_barrier()

    @pl.loop(0, NPT // IDXW)
    def _(j):
        pltpu.sync_copy(acc.at[pl.ds(base + j * IDXW, IDXW)],
                        out_hbm.at[cid].at[pl.ds(base + j * IDXW, IDXW)])

    pltpu.sync_copy(acc.at[pl.ds(base + NPT - rem, rem)],
                    out_hbm.at[cid].at[pl.ds(base + NPT - rem, rem)])


def _scatter(msg, dst):
    kfn = pl.kernel(
        _scatter_body,
        out_type=jax.ShapeDtypeStruct((NC, N, TW), jnp.float32),
        mesh=_mesh,
        scratch_types=[
            pltpu.VMEM((1, IDXW), jnp.int32),
            pltpu.VMEM((IDXW, TW), jnp.float32),
            pltpu.VMEM_SHARED((N, TW), jnp.float32),
        ],
        compiler_params=_sc_params,
    )
    return kfn(msg, dst)


def _node_body(feat_ref, coord_ref, vel_ref, part_ref,
               wn1a_ref, wn1b_ref, bn1_ref, wn2_ref, bn2_ref,
               wv1_ref, bv1_ref, wv2_ref, bv2_ref, h_ref, x_ref):
    f = feat_ref[...]
    p = part_ref[...]
    agg = p[0] + p[1]
    h_agg = agg[:, :F]
    x_agg = agg[:, F:F + 3]
    pre_v = jnp.dot(f, wv1_ref[...], preferred_element_type=jnp.float32) + bv1_ref[...]
    hv = pre_v * jax.nn.sigmoid(pre_v)
    vcoef = jnp.dot(hv, wv2_ref[...], preferred_element_type=jnp.float32) + bv2_ref[0, 0]
    pre1 = (jnp.dot(f, wn1a_ref[...], preferred_element_type=jnp.float32)
            + jnp.dot(h_agg, wn1b_ref[...], preferred_element_type=jnp.float32)
            + bn1_ref[...])
    h1 = pre1 * jax.nn.sigmoid(pre1)
    h_ref[...] = jnp.dot(h1, wn2_ref[...], preferred_element_type=jnp.float32) + bn2_ref[...]
    x_ref[...] = coord_ref[...] + vcoef * vel_ref[...] + x_agg


def _node_mlp(feat, coordinate, velocity, part,
              wn1a, wn1b, bn1, wn2, bn2, wv1, bv1, wv2, bv2):
    full = lambda shape: pl.BlockSpec(shape, lambda i: tuple(0 for _ in shape))
    return pl.pallas_call(
        _node_body,
        grid=(N // NB,),
        in_specs=[
            pl.BlockSpec((NB, F), lambda i: (i, 0)),
            pl.BlockSpec((NB, 3), lambda i: (i, 0)),
            pl.BlockSpec((NB, 3), lambda i: (i, 0)),
            pl.BlockSpec((NC, NB, TW), lambda i: (0, i, 0)),
            full((F, F)), full((F, F)), full((1, F)), full((F, F)),
            full((1, F)), full((F, F)), full((1, F)), full((F, 1)),
            full((1, 1)),
        ],
        out_specs=[
            pl.BlockSpec((NB, F), lambda i: (i, 0)),
            pl.BlockSpec((NB, 3), lambda i: (i, 0)),
        ],
        out_shape=[
            jax.ShapeDtypeStruct((N, F), jnp.float32),
            jax.ShapeDtypeStruct((N, 3), jnp.float32),
        ],
    )(feat, coordinate, velocity, part,
      wn1a, wn1b, bn1, wn2, bn2, wv1, bv1, wv2, bv2)


def kernel(feat, coordinate, velocity, edge_index,
           W_e1, b_e1, W_e2, b_e2,
           W_c1, b_c1, W_c2, b_c2,
           W_n1, b_n1, W_n2, b_n2,
           W_v1, b_v1, W_v2, b_v2):
    src = edge_index[0].reshape(IDXROWS, IDXW)
    dst = edge_index[1].reshape(IDXROWS, IDXW)
    wa = W_e1[:F]
    wb = W_e1[F:2 * F]
    wd2 = W_e1[2 * F:2 * F + 1]

    ta, tb = _build_tables(feat, coordinate, wa, wb)
    s = _gather(ta, tb, src, dst)
    msg = _edge_mlp(s, b_e1.reshape(1, F), W_e2, b_e2.reshape(1, F),
                    W_c1, b_c1.reshape(1, F), W_c2, b_c2.reshape(1, 1), wd2)
    part = _scatter(msg, dst)
    h_new, x_new = _node_mlp(
        feat, coordinate, velocity, part,
        W_n1[:F], W_n1[F:], b_n1.reshape(1, F), W_n2, b_n2.reshape(1, F),
        W_v1, b_v1.reshape(1, F), W_v2, b_v2.reshape(1, 1))
    return (h_new, x_new)
